# baseline (speedup 1.0000x reference)
import functools

import jax
import jax.numpy as jnp
from jax import lax
from jax.experimental import pallas as pl
from jax.experimental.pallas import tpu as pltpu
from jax.experimental.pallas import tpu_sc as plsc

NUM_USERS = 1000000
EMBED_DIM = 64
BATCH = 16384
FEAT_DIM = 128

_NC = 2
_NS = 16
_NW = _NC * _NS
_B_PER_W = BATCH // _NW
_CHUNK = 128
_N_CHUNKS = _B_PER_W // _CHUNK


def _sc_gather_kernel(table_hbm, idx_hbm, out_hbm, idx_v, rows_v, sem):
    wid = lax.axis_index("s") * _NC + lax.axis_index("c")
    base = wid * _B_PER_W
    pltpu.sync_copy(idx_hbm.at[pl.ds(wid * _N_CHUNKS, _N_CHUNKS)], idx_v)
    copies = [
        pltpu.async_copy(
            table_hbm.at[idx_v.at[j]],
            rows_v.at[pl.ds(j * _CHUNK, _CHUNK)],
            sem,
        )
        for j in range(_N_CHUNKS)
    ]
    for c in copies:
        c.wait()
    pltpu.sync_copy(rows_v, out_hbm.at[pl.ds(base, _B_PER_W)])


@jax.jit
def _sc_gather(table, idx2d):
    mesh = plsc.VectorSubcoreMesh(core_axis_name="c", subcore_axis_name="s")
    return pl.kernel(
        _sc_gather_kernel,
        mesh=mesh,
        compiler_params=pltpu.CompilerParams(use_tc_tiling_on_sc=False),
        out_type=jax.ShapeDtypeStruct((BATCH, EMBED_DIM), jnp.float32),
        scratch_types=[
            pltpu.VMEM((_N_CHUNKS, _CHUNK), jnp.int32),
            pltpu.VMEM((_B_PER_W, EMBED_DIM), jnp.float32),
            pltpu.SemaphoreType.DMA,
        ],
    )(table, idx2d)


_BLK = 2048


def _tc_fused_kernel(uvec_ref, feats_ref, w_ref, b_ref, out_ref):
    emb = (
        jnp.dot(feats_ref[...], w_ref[...], preferred_element_type=jnp.float32)
        + b_ref[...]
    )
    dot = jnp.sum(uvec_ref[...] * emb, axis=1, keepdims=True)
    out_ref[...] = jax.nn.sigmoid(dot)


@jax.jit
def _tc_fused(uvec, feats, w, b2d):
    grid = (BATCH // _BLK,)
    return pl.pallas_call(
        _tc_fused_kernel,
        grid=grid,
        in_specs=[
            pl.BlockSpec((_BLK, EMBED_DIM), lambda i: (i, 0)),
            pl.BlockSpec((_BLK, FEAT_DIM), lambda i: (i, 0)),
            pl.BlockSpec((FEAT_DIM, EMBED_DIM), lambda i: (0, 0)),
            pl.BlockSpec((1, EMBED_DIM), lambda i: (0, 0)),
        ],
        out_specs=pl.BlockSpec((_BLK, 1), lambda i: (i, 0)),
        out_shape=jax.ShapeDtypeStruct((BATCH, 1), jnp.float32),
    )(uvec, feats, w, b2d)


def kernel(user_ids, restaurant_features, user_embedding_table, dense_kernel, dense_bias):
    idx2d = user_ids.astype(jnp.int32).reshape(BATCH // _CHUNK, _CHUNK)
    uvec = _sc_gather(user_embedding_table, idx2d)
    return _tc_fused(
        uvec,
        restaurant_features,
        dense_kernel,
        dense_bias.reshape(1, EMBED_DIM),
    )


# speedup vs baseline: 1.6819x; 1.6819x over previous
import functools

import jax
import jax.numpy as jnp
from jax import lax
from jax.experimental import pallas as pl
from jax.experimental.pallas import tpu as pltpu
from jax.experimental.pallas import tpu_sc as plsc

NUM_USERS = 1000000
EMBED_DIM = 64
BATCH = 16384
FEAT_DIM = 128

_NC = 2
_NS = 16
_NW = _NC * _NS
_B_PER_W = BATCH // _NW
_LANES = 16
_GROUPS = _B_PER_W // _LANES
_DEPTH = 4


def _sc_gather_kernel(table_hbm, idx_hbm, out_hbm, idx_v, rows_v, sem):
    wid = lax.axis_index("s") * _NC + lax.axis_index("c")
    base = wid * _B_PER_W
    pltpu.sync_copy(idx_hbm.at[pl.ds(base, _B_PER_W)], idx_v)
    lane = lax.iota(jnp.int32, _LANES)

    def issue_group(g):
        v = idx_v[pl.ds(g * _LANES, _LANES)]
        for l in range(_LANES):
            t = jnp.max(jnp.where(lane == l, v, 0))
            pltpu.async_copy(
                table_hbm.at[pl.ds(t, 1)],
                rows_v.at[pl.ds(g * _LANES + l, 1)],
                sem,
            )

    def drain_groups(g, n):
        pltpu.make_async_copy(
            table_hbm.at[pl.ds(0, n * _LANES)],
            rows_v.at[pl.ds(g * _LANES, n * _LANES)],
            sem,
        ).wait()

    for g in range(_DEPTH):
        issue_group(g)

    def body(g, carry):
        issue_group(g)
        drain_groups(g - _DEPTH, 1)
        return carry

    lax.fori_loop(_DEPTH, _GROUPS, body, 0)
    drain_groups(_GROUPS - _DEPTH, _DEPTH)
    pltpu.sync_copy(rows_v, out_hbm.at[pl.ds(base, _B_PER_W)])


@jax.jit
def _sc_gather(table, idx):
    mesh = plsc.VectorSubcoreMesh(core_axis_name="c", subcore_axis_name="s")
    return pl.kernel(
        _sc_gather_kernel,
        mesh=mesh,
        compiler_params=pltpu.CompilerParams(needs_layout_passes=False),
        out_type=jax.ShapeDtypeStruct((BATCH, EMBED_DIM), jnp.float32),
        scratch_types=[
            pltpu.VMEM((_B_PER_W,), jnp.int32),
            pltpu.VMEM((_B_PER_W, EMBED_DIM), jnp.float32),
            pltpu.SemaphoreType.DMA,
        ],
    )(table, idx)


_BLK = 2048


def _tc_fused_kernel(uvec_ref, feats_ref, w_ref, b_ref, out_ref):
    emb = (
        jnp.dot(feats_ref[...], w_ref[...], preferred_element_type=jnp.float32)
        + b_ref[...]
    )
    dot = jnp.sum(uvec_ref[...] * emb, axis=1, keepdims=True)
    out_ref[...] = jax.nn.sigmoid(dot)


@jax.jit
def _tc_fused(uvec, feats, w, b2d):
    grid = (BATCH // _BLK,)
    return pl.pallas_call(
        _tc_fused_kernel,
        grid=grid,
        in_specs=[
            pl.BlockSpec((_BLK, EMBED_DIM), lambda i: (i, 0)),
            pl.BlockSpec((_BLK, FEAT_DIM), lambda i: (i, 0)),
            pl.BlockSpec((FEAT_DIM, EMBED_DIM), lambda i: (0, 0)),
            pl.BlockSpec((1, EMBED_DIM), lambda i: (0, 0)),
        ],
        out_specs=pl.BlockSpec((_BLK, 1), lambda i: (i, 0)),
        out_shape=jax.ShapeDtypeStruct((BATCH, 1), jnp.float32),
    )(uvec, feats, w, b2d)


def kernel(user_ids, restaurant_features, user_embedding_table, dense_kernel, dense_bias):
    idx = user_ids.astype(jnp.int32).reshape(BATCH)
    uvec = _sc_gather(user_embedding_table, idx)
    return _tc_fused(
        uvec,
        restaurant_features,
        dense_kernel,
        dense_bias.reshape(1, EMBED_DIM),
    )


# speedup vs baseline: 1.6877x; 1.0034x over previous
import functools

import jax
import jax.numpy as jnp
from jax import lax
from jax.experimental import pallas as pl
from jax.experimental.pallas import tpu as pltpu
from jax.experimental.pallas import tpu_sc as plsc

NUM_USERS = 1000000
EMBED_DIM = 64
BATCH = 16384
FEAT_DIM = 128

_NC = 2
_NS = 16
_NW = _NC * _NS
_B_PER_W = BATCH // _NW
_LANES = 16
_GROUPS = _B_PER_W // _LANES
_DEPTH = 4


def _sc_gather_kernel(table_hbm, idx_hbm, out_hbm, idx_v, rows_v, sem):
    wid = lax.axis_index("s") * _NC + lax.axis_index("c")
    base = wid * _B_PER_W
    pltpu.sync_copy(idx_hbm.at[pl.ds(base, _B_PER_W)], idx_v)
    lane = lax.iota(jnp.int32, _LANES)

    def issue_group(g):
        v = idx_v[pl.ds(g * _LANES, _LANES)]
        for l in range(_LANES):
            t = jnp.max(jnp.where(lane == l, v, 0))
            pltpu.async_copy(
                table_hbm.at[pl.ds(t, 1)],
                rows_v.at[pl.ds(g * _LANES + l, 1)],
                sem,
            )

    def drain_groups(g, n):
        pltpu.make_async_copy(
            table_hbm.at[pl.ds(0, n * _LANES)],
            rows_v.at[pl.ds(g * _LANES, n * _LANES)],
            sem,
        ).wait()

    for g in range(_DEPTH):
        issue_group(g)

    def body(g, carry):
        issue_group(g)
        drain_groups(g - _DEPTH, 1)
        return carry

    lax.fori_loop(_DEPTH, _GROUPS, body, 0)
    drain_groups(_GROUPS - _DEPTH, _DEPTH)
    pltpu.sync_copy(rows_v, out_hbm.at[pl.ds(base, _B_PER_W)])


@jax.jit
def _sc_gather(table, idx):
    mesh = plsc.VectorSubcoreMesh(core_axis_name="c", subcore_axis_name="s")
    return pl.kernel(
        _sc_gather_kernel,
        mesh=mesh,
        compiler_params=pltpu.CompilerParams(
            needs_layout_passes=False, skip_device_barrier=True
        ),
        out_type=jax.ShapeDtypeStruct((BATCH, EMBED_DIM), jnp.float32),
        scratch_types=[
            pltpu.VMEM((_B_PER_W,), jnp.int32),
            pltpu.VMEM((_B_PER_W, EMBED_DIM), jnp.float32),
            pltpu.SemaphoreType.DMA,
        ],
    )(table, idx)


_BLK = 2048


def _tc_fused_kernel(uvec_ref, feats_ref, w_ref, b_ref, out_ref):
    emb = (
        jnp.dot(feats_ref[...], w_ref[...], preferred_element_type=jnp.float32)
        + b_ref[...]
    )
    dot = jnp.sum(uvec_ref[...] * emb, axis=1, keepdims=True)
    out_ref[...] = jax.nn.sigmoid(dot)


@jax.jit
def _tc_fused(uvec, feats, w, b2d):
    grid = (BATCH // _BLK,)
    return pl.pallas_call(
        _tc_fused_kernel,
        grid=grid,
        in_specs=[
            pl.BlockSpec((_BLK, EMBED_DIM), lambda i: (i, 0)),
            pl.BlockSpec((_BLK, FEAT_DIM), lambda i: (i, 0)),
            pl.BlockSpec((FEAT_DIM, EMBED_DIM), lambda i: (0, 0)),
            pl.BlockSpec((1, EMBED_DIM), lambda i: (0, 0)),
        ],
        out_specs=pl.BlockSpec((_BLK, 1), lambda i: (i, 0)),
        out_shape=jax.ShapeDtypeStruct((BATCH, 1), jnp.float32),
    )(uvec, feats, w, b2d)


def kernel(user_ids, restaurant_features, user_embedding_table, dense_kernel, dense_bias):
    idx = user_ids.astype(jnp.int32).reshape(BATCH)
    uvec = _sc_gather(user_embedding_table, idx)
    return _tc_fused(
        uvec,
        restaurant_features,
        dense_kernel,
        dense_bias.reshape(1, EMBED_DIM),
    )
